# baseline (speedup 1.0000x reference)
import functools

import jax
import jax.numpy as jnp
from jax import lax
from jax.experimental import pallas as pl
from jax.experimental.pallas import tpu as pltpu
from jax.experimental.pallas import tpu_sc as plsc

B, S, D = 4, 8192, 1024
NC, NS = 2, 16
NW = NC * NS
POS_PER_W = S // NW
C = 8
NJ = POS_PER_W // C
NB2 = NJ // 2

_VMEMS = [
    pltpu.VMEM((B, C, D), jnp.float32),
    pltpu.VMEM((B, C, D), jnp.float32),
    pltpu.VMEM((C, D), jnp.float32),
    pltpu.VMEM((C, D), jnp.float32),
]
_SEMS = [pltpu.SemaphoreType.DMA] * 6


@functools.partial(
    pl.kernel,
    mesh=plsc.VectorSubcoreMesh(core_axis_name="c", subcore_axis_name="s"),
    out_type=jax.ShapeDtypeStruct((B, S, D), jnp.float32),
    scratch_types=_VMEMS + _SEMS,
)
def _pe_add_sc(x_hbm, pe_hbm, out_hbm, xb0, xb1, peb0, peb1,
               si0, si1, so0, so1, spe0, spe1):
    xb = (xb0, xb1)
    peb = (peb0, peb1)
    si = (si0, si1)
    so = (so0, so1)
    spe = (spe0, spe1)

    wid = lax.axis_index("s") * NC + lax.axis_index("c")
    STRIDE = NW * C
    row0 = wid * C
    qmax = row0 + (NJ - 1) * STRIDE

    def start_in(p, q):
        q = pl.multiple_of(q, C)
        return pltpu.async_copy(x_hbm.at[:, pl.ds(q, C)], xb[p], si[p])

    def wait_in(p, q):
        q = pl.multiple_of(q, C)
        pltpu.make_async_copy(x_hbm.at[:, pl.ds(q, C)], xb[p], si[p]).wait()

    def start_pe(p, q):
        q = pl.multiple_of(q, C)
        return pltpu.async_copy(pe_hbm.at[pl.ds(q, C)], peb[p], spe[p])

    def wait_pe(p, q):
        q = pl.multiple_of(q, C)
        pltpu.make_async_copy(pe_hbm.at[pl.ds(q, C)], peb[p], spe[p]).wait()

    def start_out(p, q):
        q = pl.multiple_of(q, C)
        return pltpu.async_copy(xb[p], out_hbm.at[:, pl.ds(q, C)], so[p])

    def wait_out(p, q):
        q = pl.multiple_of(q, C)
        pltpu.make_async_copy(xb[p], out_hbm.at[:, pl.ds(q, C)], so[p]).wait()

    def add_pe_all(p):
        @plsc.parallel_loop(0, C, 1, unroll=1)
        def body(r):
            for g in range(D // 256):
                tv = [peb[p][r, pl.ds(g * 256 + i * 16, 16)]
                      for i in range(16)]
                for b in range(B):
                    for i in range(16):
                        plsc.addupdate(
                            xb[p].at[b, r, pl.ds(g * 256 + i * 16, 16)],
                            tv[i])

    def body(k, carry):
        q0 = row0 + (2 * k) * STRIDE
        q1 = q0 + STRIDE
        qp0 = jnp.minimum(q0 + 2 * STRIDE, qmax)

        @pl.when(k > 0)
        def _():
            wait_out(1, jnp.maximum(q1 - 2 * STRIDE, row0))
        start_in(1, q1)
        start_pe(1, q1)

        wait_pe(0, q0)
        wait_in(0, q0)
        add_pe_all(0)
        start_out(0, q0)

        wait_pe(1, q1)
        wait_in(1, q1)
        add_pe_all(1)
        start_out(1, q1)

        wait_out(0, q0)
        start_in(0, qp0)
        start_pe(0, qp0)
        return carry

    start_in(0, row0)
    start_pe(0, row0)

    lax.fori_loop(0, NB2, body, 0)

    wait_out(1, qmax)
    wait_in(0, qmax)
    wait_pe(0, qmax)


def kernel(x, pos_embed):
    return _pe_add_sc(x, pos_embed[:S])
